# Initial kernel scaffold; baseline (speedup 1.0000x reference)
import jax
import jax.numpy as jnp
from jax.experimental import pallas as pl


def kernel(input_seq, params):
    raise NotImplementedError("write your pallas kernel here")


# speedup vs baseline: 14.8745x; 14.8745x over previous
import numpy as np
import jax
import jax.numpy as jnp
from jax.experimental import pallas as pl

_HI = jax.lax.Precision.HIGHEST
_NSP = 64


def _tap_masks(offsets):
    mats = []
    for dh, dw in offsets:
        t = np.zeros((_NSP, _NSP), np.float32)
        for h in range(8):
            for w in range(8):
                hh, ww = h + dh, w + dw
                if 0 <= hh < 8 and 0 <= ww < 8:
                    t[hh * 8 + ww, h * 8 + w] = 1.0
        mats.append(t)
    return np.stack(mats)


_T3 = _tap_masks([(kh - 1, kw - 1) for kh in range(3) for kw in range(3)])
_T5 = _tap_masks([(2 * kh - 4, 2 * kw - 4) for kh in range(5) for kw in range(5)])
_PCH = np.kron(np.eye(8, dtype=np.float32), np.ones((_NSP, 1), np.float32) / _NSP)
_PSUM = np.kron(np.eye(8, dtype=np.float32), np.ones((_NSP, 1), np.float32))
_PAVG = np.tile(np.eye(_NSP, dtype=np.float32), (8, 1)) / 8.0
_E4 = np.kron(np.eye(4, dtype=np.float32), np.ones((1, _NSP), np.float32))
_E8 = np.kron(np.eye(8, dtype=np.float32), np.ones((1, _NSP), np.float32))


def _k1x1(w2d):
    return jnp.kron(w2d.T, jnp.eye(_NSP, dtype=jnp.float32))


def _dw_mats(w, taps):
    c = w.shape[0]
    wt = w.reshape(c, -1)
    return jnp.sum(wt[:, :, None, None] * jnp.asarray(taps)[None], axis=1)


def _bd(mc):
    c = mc.shape[0]
    eye = jnp.eye(c, dtype=mc.dtype)
    return (eye[:, None, :, None] * mc[:, :, None, :]).reshape(c * _NSP, c * _NSP)


def _pairswap(u):
    parts = []
    for c in range(0, 8, 2):
        parts.append(u[:, (c + 1) * _NSP:(c + 2) * _NSP])
        parts.append(u[:, c * _NSP:(c + 1) * _NSP])
    return jnp.concatenate(parts, axis=1)


def _gelu(x):
    return 0.5 * x * (1.0 + jax.lax.erf(x * np.float32(1.0 / np.sqrt(2.0))))


def _r(x):
    u = jax.lax.bitcast_convert_type(x, jnp.int32)
    lsb = jax.lax.shift_right_logical(u, 16) & 1
    u2 = (u + 0x7FFF + lsb) & jnp.int32(-65536)
    return jax.lax.bitcast_convert_type(u2, jnp.float32)


def _body(x_ref, m1, bd0, bdsp, mc1, mc2, pch, mfc1, mfc2, pavg, e4b, mcm,
          m2, mm1, bdm, mm2, mqkv, bdq, psum, e8b, mpo, lmat, v512, v1536,
          sp, o_ref):
    def dot(a, b):
        return jax.lax.dot(a, b, precision=_HI)

    def rdot(a, b):
        return jax.lax.dot(_r(a), b, precision=_HI)

    def row(i):
        return v512[i:i + 1, :]

    x = x_ref[...]
    y0 = x * row(0) + row(1)
    y1 = _gelu(rdot(y0, m1[...]) + row(2))
    a1c = dot(y1, bd0[...]) + row(3)
    a2c = dot(a1c, bdsp[...]) + row(4)
    attn = jnp.concatenate([rdot(a1c, mc1[...]), rdot(a2c, mc2[...])],
                           axis=1) + row(5)
    ch = dot(attn, pch[...])
    z = jax.nn.relu(rdot(ch, mfc1[...]) * sp[1:2, 0:32] + sp[3:4, 0:32])
    ab = rdot(z, mfc2[...])
    s1 = ab[:, 0:4]
    s2 = ab[:, 4:8]
    mx = jnp.maximum(s1, s2)
    e1 = jnp.exp(s1 - mx)
    e2 = jnp.exp(s2 - mx)
    den = e1 + e2
    a1v = e1 / den
    a2v = e2 / den
    avg = dot(attn, pavg[...])
    attn1 = attn[:, 0:256]
    attn2 = attn[:, 256:512]
    mix = attn1 * dot(a1v, e4b[...]) + attn2 * dot(a2v, e4b[...])
    avg4 = jnp.concatenate([avg, avg, avg, avg], axis=1)
    am = jax.nn.sigmoid(rdot(mix * avg4, mcm[...]) + row(6))
    y2 = y1 * am
    x1 = x + row(7) * (rdot(y2, m2[...]) + row(8) + y0)
    y3 = x1 * row(9) + row(10)
    g = _gelu(dot(rdot(y3, mm1[...]) + row(11), bdm[...]) + row(12))
    x2 = x1 + row(13) * (rdot(g, mm2[...]) + row(14))
    o = x2 + x
    qkv = rdot(o, mqkv[...]) + v1536[0:1, :]
    qkv = jnp.concatenate([dot(qkv[:, 0:512], bdq[0]),
                           dot(qkv[:, 512:1024], bdq[1]),
                           dot(qkv[:, 1024:1536], bdq[2])],
                          axis=1) + v1536[1:2, :]
    q = qkv[:, 0:512]
    k = qkv[:, 512:1024]
    v = qkv[:, 1024:1536]
    sqn = 1.0 / jnp.maximum(jnp.sqrt(dot(q * q, psum[...])), 1e-12)
    skn = 1.0 / jnp.maximum(jnp.sqrt(dot(k * k, psum[...])), 1e-12)
    qh = _r(q * dot(sqn, e8b[...]))
    kh = _r(k * dot(skn, e8b[...]))
    tmp = sp[0:1, 0:8]
    same = dot(qh * kh, psum[...]) * tmp
    cross = dot(qh * _pairswap(kh), psum[...]) * tmp
    iota = jax.lax.broadcasted_iota(jnp.int32, (1, 8), 1)
    evenf = 1.0 - (iota % 2).astype(jnp.float32)
    ge = (same >= cross).astype(jnp.float32)
    gt = (same > cross).astype(jnp.float32)
    m8 = evenf * ge + (1.0 - evenf) * gt
    mb = dot(m8, e8b[...])
    vr = _r(v)
    aout = (mb * vr + (1.0 - mb) * _pairswap(vr)) * sp[0:1, 8:9]
    t = rdot(aout, mpo[...]) + row(15)
    o_ref[...] = rdot(jax.nn.relu(t), lmat[...]) + sp[2:3, 0:24]


def kernel(input_seq, params):
    p = params
    b = input_seq.shape[0]
    x = input_seq.reshape(b, 512).astype(jnp.float32)
    f32 = jnp.float32

    def w2(name):
        return p[name][:, :, 0, 0]

    def tile(v):
        return jnp.repeat(v, _NSP)

    def rr(m):
        return _r(m)

    inv_s = 1.0 / jnp.sqrt(jnp.asarray(1.0 + 1e-05, f32))
    bn1s = p['bn1_g'] * inv_s
    bn2s = p['bn2_g'] * inv_s

    m1 = rr(_k1x1(w2('proj1_w')))
    bd0 = _bd(_dw_mats(p['conv0_w'], _T3))
    bdsp = _bd(_dw_mats(p['convsp_w'], _T5))
    mc1 = rr(_k1x1(w2('conv1_w')))
    mc2 = rr(_k1x1(w2('conv2_w')))
    battn = jnp.concatenate([tile(p['conv1_b']), tile(p['conv2_b'])])
    mfc1 = rr(w2('fc1_w').T)
    mcm = rr(_k1x1(w2('conv_w')))
    mfc2 = rr(w2('fc2_w').T)
    m2 = rr(_k1x1(w2('proj2_w')))
    mm1 = rr(_k1x1(w2('mfc1_w')))
    bdm = _bd(_dw_mats(p['mdw_w'], _T3))
    mm2 = rr(_k1x1(w2('mfc2_w')))
    mqkv = rr(_k1x1(w2('qkv_w')))
    dq = _dw_mats(p['qkvdw_w'], _T3)
    bdq = jnp.stack([_bd(dq[0:8]), _bd(dq[8:16]), _bd(dq[16:24])])
    mpo = rr(_k1x1(w2('po_w')))
    lmat = rr(p['lin_w'].T)
    tempv = jnp.repeat(p['temp'].reshape(4), 2)
    asum = (p['a1'] + p['a2'] + p['a3'] + p['a4'])[0]

    v512 = jnp.zeros((16, 512), f32)
    for i, vec in enumerate([
            tile(bn1s), tile(p['bn1_b']), tile(p['proj1_b']),
            tile(p['conv0_b']), tile(p['convsp_b']), battn,
            tile(p['conv_b']), tile(p['ls1']), tile(p['proj2_b']),
            tile(bn2s), tile(p['bn2_b']), tile(p['mfc1_b']),
            tile(p['mdw_b']), tile(p['ls2']), tile(p['mfc2_b']),
            tile(p['po_b'])]):
        v512 = v512.at[i].set(vec)
    v1536 = jnp.zeros((8, 1536), f32)
    v1536 = v1536.at[0].set(tile(p['qkv_b'])).at[1].set(tile(p['qkvdw_b']))
    sp = jnp.zeros((8, 64), f32)
    sp = sp.at[0, 0:8].set(tempv).at[0, 8].set(asum)
    sp = sp.at[1, 0:32].set(p['fc1_bn_g'] * inv_s)
    sp = sp.at[3, 0:32].set(p['fc1_bn_b'])
    sp = sp.at[2, 0:24].set(p['lin_b'])

    bt = 512 if b % 512 == 0 else b
    nb = b // bt

    def const_spec(arr):
        shape = arr.shape
        if len(shape) == 2:
            return pl.BlockSpec(shape, lambda i: (0, 0))
        return pl.BlockSpec(shape, lambda i: (0, 0, 0))

    consts = [m1, bd0, bdsp, mc1, mc2, jnp.asarray(_PCH), mfc1, mfc2,
              jnp.asarray(_PAVG), jnp.asarray(_E4), mcm, m2, mm1, bdm, mm2,
              mqkv, bdq, jnp.asarray(_PSUM), jnp.asarray(_E8), mpo, lmat,
              v512, v1536, sp]

    out = pl.pallas_call(
        _body,
        grid=(nb,),
        in_specs=[pl.BlockSpec((bt, 512), lambda i: (i, 0))]
        + [const_spec(a) for a in consts],
        out_specs=pl.BlockSpec((bt, 24), lambda i: (i, 0)),
        out_shape=jax.ShapeDtypeStruct((b, 24), f32),
    )(x, *consts)
    return out


# speedup vs baseline: 27.7455x; 1.8653x over previous
import numpy as np
import jax
import jax.numpy as jnp
from jax.experimental import pallas as pl

_HI = jax.lax.Precision.HIGHEST
_NSP = 64


def _tap_masks(offsets):
    mats = []
    for dh, dw in offsets:
        t = np.zeros((_NSP, _NSP), np.float32)
        for h in range(8):
            for w in range(8):
                hh, ww = h + dh, w + dw
                if 0 <= hh < 8 and 0 <= ww < 8:
                    t[hh * 8 + ww, h * 8 + w] = 1.0
        mats.append(t)
    return np.stack(mats)


_T3 = _tap_masks([(kh - 1, kw - 1) for kh in range(3) for kw in range(3)])
_T5 = _tap_masks([(2 * kh - 4, 2 * kw - 4) for kh in range(5) for kw in range(5)])
_PCH = np.kron(np.eye(8, dtype=np.float32), np.ones((_NSP, 1), np.float32) / _NSP)
_PSUM = np.kron(np.eye(8, dtype=np.float32), np.ones((_NSP, 1), np.float32))
_PAVG = np.tile(np.eye(_NSP, dtype=np.float32), (8, 1)) / 8.0
_E4 = np.kron(np.eye(4, dtype=np.float32), np.ones((1, _NSP), np.float32))
_E8 = np.kron(np.eye(8, dtype=np.float32), np.ones((1, _NSP), np.float32))


def _k1x1(w2d):
    return jnp.kron(w2d.T, jnp.eye(_NSP, dtype=jnp.float32))


def _dw_mats(w, taps):
    c = w.shape[0]
    wt = w.reshape(c, -1)
    return jnp.sum(wt[:, :, None, None] * jnp.asarray(taps)[None], axis=1)


def _bd(mc):
    c = mc.shape[0]
    eye = jnp.eye(c, dtype=mc.dtype)
    return (eye[:, None, :, None] * mc[:, :, None, :]).reshape(c * _NSP, c * _NSP)


def _pairswap(u):
    parts = []
    for c in range(0, 8, 2):
        parts.append(u[:, (c + 1) * _NSP:(c + 2) * _NSP])
        parts.append(u[:, c * _NSP:(c + 1) * _NSP])
    return jnp.concatenate(parts, axis=1)


def _gelu(x):
    return 0.5 * x * (1.0 + jax.lax.erf(x * np.float32(1.0 / np.sqrt(2.0))))


def _r(x):
    u = jax.lax.bitcast_convert_type(x, jnp.int32)
    lsb = jax.lax.shift_right_logical(u, 16) & 1
    u2 = (u + 0x7FFF + lsb) & jnp.int32(-65536)
    return jax.lax.bitcast_convert_type(u2, jnp.float32)


def _body(x_ref, m1, bd0, bdsp, mc1, mc2, pch, mfc1, mfc2, pavg, e4b, mcm,
          m2, mm1, bdm, mm2, mqkv, bdq, psum, e8b, mpo, lmat, v512, v1536,
          sp, o_ref):
    def dot(a, b):
        return jax.lax.dot(a, b, precision=_HI)

    def rdot(a, b):
        return jax.lax.dot(a.astype(jnp.bfloat16), b,
                           preferred_element_type=jnp.float32)

    def row(i):
        return v512[i:i + 1, :]

    x = x_ref[...]
    y0 = x * row(0) + row(1)
    y1 = _gelu(rdot(y0, m1[...]) + row(2))
    a1c = dot(y1, bd0[...]) + row(3)
    a2c = dot(a1c, bdsp[...]) + row(4)
    attn = jnp.concatenate([rdot(a1c, mc1[...]), rdot(a2c, mc2[...])],
                           axis=1) + row(5)
    ch = dot(attn, pch[...])
    z = jax.nn.relu(rdot(ch, mfc1[...]) * sp[1:2, 0:32] + sp[3:4, 0:32])
    ab = rdot(z, mfc2[...])
    s1 = ab[:, 0:4]
    s2 = ab[:, 4:8]
    mx = jnp.maximum(s1, s2)
    e1 = jnp.exp(s1 - mx)
    e2 = jnp.exp(s2 - mx)
    den = e1 + e2
    a1v = e1 / den
    a2v = e2 / den
    avg = dot(attn, pavg[...])
    attn1 = attn[:, 0:256]
    attn2 = attn[:, 256:512]
    mix = attn1 * dot(a1v, e4b[...]) + attn2 * dot(a2v, e4b[...])
    avg4 = jnp.concatenate([avg, avg, avg, avg], axis=1)
    am = jax.nn.sigmoid(rdot(mix * avg4, mcm[...]) + row(6))
    y2 = y1 * am
    x1 = x + row(7) * (rdot(y2, m2[...]) + row(8) + y0)
    y3 = x1 * row(9) + row(10)
    g = _gelu(dot(rdot(y3, mm1[...]) + row(11), bdm[...]) + row(12))
    x2 = x1 + row(13) * (rdot(g, mm2[...]) + row(14))
    o = x2 + x
    qkv = rdot(o, mqkv[...]) + v1536[0:1, :]
    qkv = jnp.concatenate([dot(qkv[:, 0:512], bdq[0]),
                           dot(qkv[:, 512:1024], bdq[1]),
                           dot(qkv[:, 1024:1536], bdq[2])],
                          axis=1) + v1536[1:2, :]
    q = qkv[:, 0:512]
    k = qkv[:, 512:1024]
    v = qkv[:, 1024:1536]
    sqn = 1.0 / jnp.maximum(jnp.sqrt(dot(q * q, psum[...])), 1e-12)
    skn = 1.0 / jnp.maximum(jnp.sqrt(dot(k * k, psum[...])), 1e-12)
    qh = _r(q * dot(sqn, e8b[...]))
    kh = _r(k * dot(skn, e8b[...]))
    tmp = sp[0:1, 0:8]
    same = dot(qh * kh, psum[...]) * tmp
    cross = dot(qh * _pairswap(kh), psum[...]) * tmp
    iota = jax.lax.broadcasted_iota(jnp.int32, (1, 8), 1)
    evenf = 1.0 - (iota % 2).astype(jnp.float32)
    ge = (same >= cross).astype(jnp.float32)
    gt = (same > cross).astype(jnp.float32)
    m8 = evenf * ge + (1.0 - evenf) * gt
    mb = dot(m8, e8b[...])
    vr = _r(v)
    aout = (mb * vr + (1.0 - mb) * _pairswap(vr)) * sp[0:1, 8:9]
    t = rdot(aout, mpo[...]) + row(15)
    o_ref[...] = rdot(jax.nn.relu(t), lmat[...]) + sp[2:3, 0:24]


def kernel(input_seq, params):
    p = params
    b = input_seq.shape[0]
    x = input_seq.reshape(b, 512).astype(jnp.float32)
    f32 = jnp.float32

    def w2(name):
        return p[name][:, :, 0, 0]

    def tile(v):
        return jnp.repeat(v, _NSP)

    def rr(m):
        return m.astype(jnp.bfloat16)

    inv_s = 1.0 / jnp.sqrt(jnp.asarray(1.0 + 1e-05, f32))
    bn1s = p['bn1_g'] * inv_s
    bn2s = p['bn2_g'] * inv_s

    m1 = rr(_k1x1(w2('proj1_w')))
    bd0 = _bd(_dw_mats(p['conv0_w'], _T3))
    bdsp = _bd(_dw_mats(p['convsp_w'], _T5))
    mc1 = rr(_k1x1(w2('conv1_w')))
    mc2 = rr(_k1x1(w2('conv2_w')))
    battn = jnp.concatenate([tile(p['conv1_b']), tile(p['conv2_b'])])
    mfc1 = rr(w2('fc1_w').T)
    mcm = rr(_k1x1(w2('conv_w')))
    mfc2 = rr(w2('fc2_w').T)
    m2 = rr(_k1x1(w2('proj2_w')))
    mm1 = rr(_k1x1(w2('mfc1_w')))
    bdm = _bd(_dw_mats(p['mdw_w'], _T3))
    mm2 = rr(_k1x1(w2('mfc2_w')))
    mqkv = rr(_k1x1(w2('qkv_w')))
    dq = _dw_mats(p['qkvdw_w'], _T3)
    bdq = jnp.stack([_bd(dq[0:8]), _bd(dq[8:16]), _bd(dq[16:24])])
    mpo = rr(_k1x1(w2('po_w')))
    lmat = rr(p['lin_w'].T)
    tempv = jnp.repeat(p['temp'].reshape(4), 2)
    asum = (p['a1'] + p['a2'] + p['a3'] + p['a4'])[0]

    v512 = jnp.zeros((16, 512), f32)
    for i, vec in enumerate([
            tile(bn1s), tile(p['bn1_b']), tile(p['proj1_b']),
            tile(p['conv0_b']), tile(p['convsp_b']), battn,
            tile(p['conv_b']), tile(p['ls1']), tile(p['proj2_b']),
            tile(bn2s), tile(p['bn2_b']), tile(p['mfc1_b']),
            tile(p['mdw_b']), tile(p['ls2']), tile(p['mfc2_b']),
            tile(p['po_b'])]):
        v512 = v512.at[i].set(vec)
    v1536 = jnp.zeros((8, 1536), f32)
    v1536 = v1536.at[0].set(tile(p['qkv_b'])).at[1].set(tile(p['qkvdw_b']))
    sp = jnp.zeros((8, 64), f32)
    sp = sp.at[0, 0:8].set(tempv).at[0, 8].set(asum)
    sp = sp.at[1, 0:32].set(p['fc1_bn_g'] * inv_s)
    sp = sp.at[3, 0:32].set(p['fc1_bn_b'])
    sp = sp.at[2, 0:24].set(p['lin_b'])

    bt = 512 if b % 512 == 0 else b
    nb = b // bt

    def const_spec(arr):
        shape = arr.shape
        if len(shape) == 2:
            return pl.BlockSpec(shape, lambda i: (0, 0))
        return pl.BlockSpec(shape, lambda i: (0, 0, 0))

    consts = [m1, bd0, bdsp, mc1, mc2, jnp.asarray(_PCH), mfc1, mfc2,
              jnp.asarray(_PAVG), jnp.asarray(_E4), mcm, m2, mm1, bdm, mm2,
              mqkv, bdq, jnp.asarray(_PSUM), jnp.asarray(_E8), mpo, lmat,
              v512, v1536, sp]

    out = pl.pallas_call(
        _body,
        grid=(nb,),
        in_specs=[pl.BlockSpec((bt, 512), lambda i: (i, 0))]
        + [const_spec(a) for a in consts],
        out_specs=pl.BlockSpec((bt, 24), lambda i: (i, 0)),
        out_shape=jax.ShapeDtypeStruct((b, 24), f32),
    )(x, *consts)
    return out


# speedup vs baseline: 33.2673x; 1.1990x over previous
import numpy as np
import jax
import jax.numpy as jnp
from jax.experimental import pallas as pl

_HI = jax.lax.Precision.HIGHEST
_NSP = 64


def _tap_masks(offsets):
    mats = []
    for dh, dw in offsets:
        t = np.zeros((_NSP, _NSP), np.float32)
        for h in range(8):
            for w in range(8):
                hh, ww = h + dh, w + dw
                if 0 <= hh < 8 and 0 <= ww < 8:
                    t[hh * 8 + ww, h * 8 + w] = 1.0
        mats.append(t)
    return np.stack(mats)


_T3 = _tap_masks([(kh - 1, kw - 1) for kh in range(3) for kw in range(3)])
_T5 = _tap_masks([(2 * kh - 4, 2 * kw - 4) for kh in range(5) for kw in range(5)])
_PCH = np.kron(np.eye(8, dtype=np.float32), np.ones((_NSP, 1), np.float32) / _NSP)
_PSUM = np.kron(np.eye(8, dtype=np.float32), np.ones((_NSP, 1), np.float32))
_PAVG = np.tile(np.eye(_NSP, dtype=np.float32), (8, 1)) / 8.0
_E4 = np.kron(np.eye(4, dtype=np.float32), np.ones((1, _NSP), np.float32))
_E8 = np.kron(np.eye(8, dtype=np.float32), np.ones((1, _NSP), np.float32))


def _k1x1(w2d):
    return jnp.kron(w2d.T, jnp.eye(_NSP, dtype=jnp.float32))


def _dw_mats(w, taps):
    c = w.shape[0]
    wt = w.reshape(c, -1)
    return jnp.sum(wt[:, :, None, None] * jnp.asarray(taps)[None], axis=1)


def _bd(mc):
    c = mc.shape[0]
    eye = jnp.eye(c, dtype=mc.dtype)
    return (eye[:, None, :, None] * mc[:, :, None, :]).reshape(c * _NSP, c * _NSP)


def _pairswap(u):
    parts = []
    for c in range(0, 8, 2):
        parts.append(u[:, (c + 1) * _NSP:(c + 2) * _NSP])
        parts.append(u[:, c * _NSP:(c + 1) * _NSP])
    return jnp.concatenate(parts, axis=1)


def _gelu(x):
    return 0.5 * x * (1.0 + jax.lax.erf(x * np.float32(1.0 / np.sqrt(2.0))))


def _r(x):
    u = jax.lax.bitcast_convert_type(x, jnp.int32)
    lsb = jax.lax.shift_right_logical(u, 16) & 1
    u2 = (u + 0x7FFF + lsb) & jnp.int32(-65536)
    return jax.lax.bitcast_convert_type(u2, jnp.float32)


def _body(x_ref, m1, bd0, bdsp, mc1, mc2, pch, mfc1, mfc2, pavg, e4b, mcm,
          m2, mm1, bdm, mm2, mqkv, bdq, psum, e8b, mpo, lmat, v512, v1536,
          sp, o_ref):
    def dot(a, b):
        return jax.lax.dot(a, b, precision=_HI)

    def hdot(a, bhl):
        f = jnp.float32
        bhi, blo = bhl[0], bhl[1]
        ahi = a.astype(jnp.bfloat16)
        alo = (a - ahi.astype(f)).astype(jnp.bfloat16)
        return (jax.lax.dot(ahi, bhi, preferred_element_type=f)
                + (jax.lax.dot(ahi, blo, preferred_element_type=f)
                   + jax.lax.dot(alo, bhi, preferred_element_type=f)))

    def rdot(a, b):
        return jax.lax.dot(a.astype(jnp.bfloat16), b,
                           preferred_element_type=jnp.float32)

    def row(i):
        return v512[i:i + 1, :]

    x = x_ref[...]
    y0 = x * row(0) + row(1)
    y1 = _gelu(rdot(y0, m1[...]) + row(2))
    a1c = hdot(y1, bd0[...]) + row(3)
    a2c = hdot(a1c, bdsp[...]) + row(4)
    attn = jnp.concatenate([rdot(a1c, mc1[...]), rdot(a2c, mc2[...])],
                           axis=1) + row(5)
    ch = dot(attn, pch[...])
    z = jax.nn.relu(rdot(ch, mfc1[...]) * sp[1:2, 0:32] + sp[3:4, 0:32])
    ab = rdot(z, mfc2[...])
    s1 = ab[:, 0:4]
    s2 = ab[:, 4:8]
    mx = jnp.maximum(s1, s2)
    e1 = jnp.exp(s1 - mx)
    e2 = jnp.exp(s2 - mx)
    den = e1 + e2
    a1v = e1 / den
    a2v = e2 / den
    avg = dot(attn, pavg[...])
    attn1 = attn[:, 0:256]
    attn2 = attn[:, 256:512]
    mix = attn1 * dot(a1v, e4b[...]) + attn2 * dot(a2v, e4b[...])
    avg4 = jnp.concatenate([avg, avg, avg, avg], axis=1)
    am = jax.nn.sigmoid(rdot(mix * avg4, mcm[...]) + row(6))
    y2 = y1 * am
    x1 = x + row(7) * (rdot(y2, m2[...]) + row(8) + y0)
    y3 = x1 * row(9) + row(10)
    g = _gelu(hdot(rdot(y3, mm1[...]) + row(11), bdm[...]) + row(12))
    x2 = x1 + row(13) * (rdot(g, mm2[...]) + row(14))
    o = x2 + x
    qkv = rdot(o, mqkv[...]) + v1536[0:1, :]
    qkv = jnp.concatenate([hdot(qkv[:, 0:512], bdq[:, 0]),
                           hdot(qkv[:, 512:1024], bdq[:, 1]),
                           hdot(qkv[:, 1024:1536], bdq[:, 2])],
                          axis=1) + v1536[1:2, :]
    q = qkv[:, 0:512]
    k = qkv[:, 512:1024]
    v = qkv[:, 1024:1536]
    sqn = 1.0 / jnp.maximum(jnp.sqrt(dot(q * q, psum[...])), 1e-12)
    skn = 1.0 / jnp.maximum(jnp.sqrt(dot(k * k, psum[...])), 1e-12)
    qh = _r(q * dot(sqn, e8b[...]))
    kh = _r(k * dot(skn, e8b[...]))
    tmp = sp[0:1, 0:8]
    same = dot(qh * kh, psum[...]) * tmp
    cross = dot(qh * _pairswap(kh), psum[...]) * tmp
    iota = jax.lax.broadcasted_iota(jnp.int32, (1, 8), 1)
    evenf = 1.0 - (iota % 2).astype(jnp.float32)
    ge = (same >= cross).astype(jnp.float32)
    gt = (same > cross).astype(jnp.float32)
    m8 = evenf * ge + (1.0 - evenf) * gt
    mb = dot(m8, e8b[...])
    vr = _r(v)
    aout = (mb * vr + (1.0 - mb) * _pairswap(vr)) * sp[0:1, 8:9]
    t = rdot(aout, mpo[...]) + row(15)
    o_ref[...] = rdot(jax.nn.relu(t), lmat[...]) + sp[2:3, 0:24]


def kernel(input_seq, params):
    p = params
    b = input_seq.shape[0]
    x = input_seq.reshape(b, 512).astype(jnp.float32)
    f32 = jnp.float32

    def w2(name):
        return p[name][:, :, 0, 0]

    def tile(v):
        return jnp.repeat(v, _NSP)

    def rr(m):
        return m.astype(jnp.bfloat16)

    inv_s = 1.0 / jnp.sqrt(jnp.asarray(1.0 + 1e-05, f32))
    bn1s = p['bn1_g'] * inv_s
    bn2s = p['bn2_g'] * inv_s

    def hl(m):
        hi = m.astype(jnp.bfloat16)
        lo = (m - hi.astype(f32)).astype(jnp.bfloat16)
        return jnp.stack([hi, lo])

    m1 = rr(_k1x1(w2('proj1_w')))
    bd0 = hl(_bd(_dw_mats(p['conv0_w'], _T3)))
    bdsp = hl(_bd(_dw_mats(p['convsp_w'], _T5)))
    mc1 = rr(_k1x1(w2('conv1_w')))
    mc2 = rr(_k1x1(w2('conv2_w')))
    battn = jnp.concatenate([tile(p['conv1_b']), tile(p['conv2_b'])])
    mfc1 = rr(w2('fc1_w').T)
    mcm = rr(_k1x1(w2('conv_w')))
    mfc2 = rr(w2('fc2_w').T)
    m2 = rr(_k1x1(w2('proj2_w')))
    mm1 = rr(_k1x1(w2('mfc1_w')))
    bdm = hl(_bd(_dw_mats(p['mdw_w'], _T3)))
    mm2 = rr(_k1x1(w2('mfc2_w')))
    mqkv = rr(_k1x1(w2('qkv_w')))
    dq = _dw_mats(p['qkvdw_w'], _T3)
    bdq = jnp.stack([hl(_bd(dq[0:8])), hl(_bd(dq[8:16])),
                     hl(_bd(dq[16:24]))], axis=1)
    mpo = rr(_k1x1(w2('po_w')))
    lmat = rr(p['lin_w'].T)
    tempv = jnp.repeat(p['temp'].reshape(4), 2)
    asum = (p['a1'] + p['a2'] + p['a3'] + p['a4'])[0]

    v512 = jnp.zeros((16, 512), f32)
    for i, vec in enumerate([
            tile(bn1s), tile(p['bn1_b']), tile(p['proj1_b']),
            tile(p['conv0_b']), tile(p['convsp_b']), battn,
            tile(p['conv_b']), tile(p['ls1']), tile(p['proj2_b']),
            tile(bn2s), tile(p['bn2_b']), tile(p['mfc1_b']),
            tile(p['mdw_b']), tile(p['ls2']), tile(p['mfc2_b']),
            tile(p['po_b'])]):
        v512 = v512.at[i].set(vec)
    v1536 = jnp.zeros((8, 1536), f32)
    v1536 = v1536.at[0].set(tile(p['qkv_b'])).at[1].set(tile(p['qkvdw_b']))
    sp = jnp.zeros((8, 64), f32)
    sp = sp.at[0, 0:8].set(tempv).at[0, 8].set(asum)
    sp = sp.at[1, 0:32].set(p['fc1_bn_g'] * inv_s)
    sp = sp.at[3, 0:32].set(p['fc1_bn_b'])
    sp = sp.at[2, 0:24].set(p['lin_b'])

    bt = 512 if b % 512 == 0 else b
    nb = b // bt

    def const_spec(arr):
        shape = arr.shape
        return pl.BlockSpec(shape, lambda i: (0,) * len(shape))

    consts = [m1, bd0, bdsp, mc1, mc2, jnp.asarray(_PCH), mfc1, mfc2,
              jnp.asarray(_PAVG), jnp.asarray(_E4), mcm, m2, mm1, bdm, mm2,
              mqkv, bdq, jnp.asarray(_PSUM), jnp.asarray(_E8), mpo, lmat,
              v512, v1536, sp]

    out = pl.pallas_call(
        _body,
        grid=(nb,),
        in_specs=[pl.BlockSpec((bt, 512), lambda i: (i, 0))]
        + [const_spec(a) for a in consts],
        out_specs=pl.BlockSpec((bt, 24), lambda i: (i, 0)),
        out_shape=jax.ShapeDtypeStruct((b, 24), f32),
    )(x, *consts)
    return out
